# Initial kernel scaffold; baseline (speedup 1.0000x reference)
import jax
import jax.numpy as jnp
from jax.experimental import pallas as pl


def kernel(x, edge_index, batch, W1, b1, W2, b2):
    raise NotImplementedError("write your pallas kernel here")


# speedup vs baseline: 18.3648x; 18.3648x over previous
import dataclasses
import functools

import jax
import jax.numpy as jnp
from jax import lax
from jax.experimental import pallas as pl
from jax.experimental.pallas import tpu as pltpu
from jax.experimental.pallas import tpu_sc as plsc

NC, NS, L = 2, 16, 16


def _sc_compiler_params():
    cp = pltpu.CompilerParams()
    if "needs_layout_passes" in pltpu.CompilerParams.__dataclass_fields__:
        cp = dataclasses.replace(cp, needs_layout_passes=False)
    return cp


def _make_deg_kernel(E, N):
    NW = NC * NS
    EPW = E // NW
    NV = EPW // L
    REM = EPW - NV * L
    mesh = plsc.VectorSubcoreMesh(core_axis_name="c", subcore_axis_name="s")

    @functools.partial(
        pl.kernel,
        out_type=jax.ShapeDtypeStruct((NW, N), jnp.float32),
        mesh=mesh,
        compiler_params=_sc_compiler_params(),
        scratch_types=[
            pltpu.VMEM((EPW + L,), jnp.int32),
            pltpu.VMEM((N,), jnp.float32),
        ],
    )
    def deg_kernel(dst_hbm, out_hbm, idx_v, hist_v):
        wid = lax.axis_index("s") * NC + lax.axis_index("c")
        base = wid * EPW
        pltpu.sync_copy(dst_hbm.at[pl.ds(base, EPW)], idx_v.at[pl.ds(0, EPW)])
        zf = jnp.zeros((L,), jnp.float32)
        idx_v[pl.ds(EPW, L)] = jnp.zeros((L,), jnp.int32)

        @pl.loop(0, N, step=L)
        def _(i):
            hist_v[pl.ds(i, L)] = zf

        ones = jnp.ones((L,), jnp.float32)

        @pl.loop(0, NV * L, step=L)
        def _(i):
            plsc.addupdate_scatter(hist_v, [idx_v[pl.ds(i, L)]], ones)

        if REM:
            mask = lax.iota(jnp.int32, L) < REM
            plsc.addupdate_scatter(hist_v, [idx_v[pl.ds(NV * L, L)]], ones,
                                   mask=mask)
        pltpu.sync_copy(hist_v, out_hbm.at[wid])

    return deg_kernel


def _make_agg_kernel(E, N, F):
    EPS = E // NS
    C = 128
    NCH = EPS // C
    REM = EPS - NCH * C
    RPW = ((N + NS - 1) // NS + 7) // 8 * 8
    RPW_LAST = N - RPW * (NS - 1)
    assert RPW_LAST > 0 and RPW_LAST % 8 == 0
    mesh = plsc.VectorSubcoreMesh(core_axis_name="c", subcore_axis_name="s")

    @functools.partial(
        pl.kernel,
        out_type=[
            jax.ShapeDtypeStruct((N, F), jnp.float32),
            jax.ShapeDtypeStruct((N, F), jnp.float32),
        ],
        mesh=mesh,
        compiler_params=_sc_compiler_params(),
        scratch_types=[
            pltpu.VMEM((C,), jnp.int32),
            pltpu.VMEM((C,), jnp.int32),
            pltpu.VMEM((C, F), jnp.float32),
            pltpu.VMEM((REM,), jnp.int32) if REM else pltpu.VMEM((L,), jnp.int32),
            pltpu.VMEM((REM,), jnp.int32) if REM else pltpu.VMEM((L,), jnp.int32),
            pltpu.VMEM((REM, F), jnp.float32) if REM else pltpu.VMEM((L, F), jnp.float32),
            pltpu.VMEM_SHARED((N, F), jnp.float32),
            pltpu.SemaphoreType.DMA,
        ],
    )
    def agg_kernel(xs_a, xs_b, src_hbm, dst_hbm, zero_hbm, agg_a, agg_b,
                   sidx, didx, rows, sidx_r, didx_r, rows_r, acc, sem):
        cid = lax.axis_index("c")
        sid = lax.axis_index("s")
        roff = pl.multiple_of(sid * RPW, 8)

        @pl.when(sid < NS - 1)
        def _():
            pltpu.sync_copy(zero_hbm.at[pl.ds(roff, RPW)],
                            acc.at[pl.ds(roff, RPW)])

        @pl.when(sid == NS - 1)
        def _():
            loff = pl.multiple_of((NS - 1) * RPW, 8)
            pltpu.sync_copy(zero_hbm.at[pl.ds(loff, RPW_LAST)],
                            acc.at[pl.ds(loff, RPW_LAST)])

        plsc.subcore_barrier()

        def run(table, out):
            base = sid * EPS

            @pl.loop(0, NCH)
            def _(i):
                off = base + i * C
                pltpu.sync_copy(src_hbm.at[pl.ds(off, C)], sidx)
                pltpu.sync_copy(dst_hbm.at[pl.ds(off, C)], didx)
                pltpu.async_copy(table.at[sidx], rows, sem).wait()
                pltpu.sync_copy(rows, acc.at[didx], add=True)

            if REM:
                off = base + NCH * C
                pltpu.sync_copy(src_hbm.at[pl.ds(off, REM)], sidx_r)
                pltpu.sync_copy(dst_hbm.at[pl.ds(off, REM)], didx_r)
                pltpu.async_copy(table.at[sidx_r], rows_r, sem).wait()
                pltpu.sync_copy(rows_r, acc.at[didx_r], add=True)
            plsc.subcore_barrier()

            @pl.when(sid < NS - 1)
            def _():
                pltpu.sync_copy(acc.at[pl.ds(roff, RPW)],
                                out.at[pl.ds(roff, RPW)])

            @pl.when(sid == NS - 1)
            def _():
                loff = pl.multiple_of((NS - 1) * RPW, 8)
                pltpu.sync_copy(acc.at[pl.ds(loff, RPW_LAST)],
                                out.at[pl.ds(loff, RPW_LAST)])

        @pl.when(cid == 0)
        def _():
            run(xs_a, agg_a)

        @pl.when(cid == 1)
        def _():
            run(xs_b, agg_b)

    return agg_kernel


def _dinv_call(degp, N):
    def body(degp_ref, dinv_ref):
        deg = jnp.sum(degp_ref[...], axis=0, keepdims=True) + 1.0
        dinv_ref[...] = lax.rsqrt(deg)

    return pl.pallas_call(
        body, out_shape=jax.ShapeDtypeStruct((1, N), jnp.float32))(degp)


def _prologue_call(x, dinv_col, N, F):
    def body(x_ref, dv_ref, a_ref, b_ref):
        xs = x_ref[...] * dv_ref[...]
        a_ref[...] = xs[:, :F]
        b_ref[...] = xs[:, F:]

    return pl.pallas_call(
        body,
        out_shape=[jax.ShapeDtypeStruct((N, F), jnp.float32),
                   jax.ShapeDtypeStruct((N, F), jnp.float32)])(x, dinv_col)


def _epilogue_call(x, agg_a, agg_b, dinv_col, batch3, W1, b1, W2, b2,
                   N, R, G, HID):
    nblk = N // R

    def body(x_ref, aa_ref, ab_ref, dv_ref, b_ref, W1_ref, b1_ref, W2_ref,
             b2_ref, node_ref, graph_ref, sums_ref, cnts_ref):
        i = pl.program_id(0)
        dv = dv_ref[...]
        agg = jnp.concatenate([aa_ref[...], ab_ref[...]], axis=1)
        Z = dv * (agg + dv * x_ref[...])
        H = jnp.tanh(
            jnp.dot(Z, W1_ref[...], preferred_element_type=jnp.float32)
            + b1_ref[...])
        node_ref[...] = H
        bat = b_ref[0]
        gid = lax.broadcasted_iota(jnp.int32, (G, R), 0)
        onehot = (bat == gid).astype(jnp.float32)
        psum = jnp.dot(onehot, H, preferred_element_type=jnp.float32)
        pcnt = jnp.sum(onehot, axis=1, keepdims=True)

        @pl.when(i == 0)
        def _():
            sums_ref[...] = psum
            cnts_ref[...] = jnp.broadcast_to(pcnt, (G, 128))

        @pl.when(i > 0)
        def _():
            sums_ref[...] += psum
            cnts_ref[...] += jnp.broadcast_to(pcnt, (G, 128))

        @pl.when(i == nblk - 1)
        def _():
            cnt = jnp.maximum(cnts_ref[:, :1], 1.0)
            mean = sums_ref[...] / cnt
            graph_ref[...] = jnp.tanh(
                jnp.dot(mean, W2_ref[...], preferred_element_type=jnp.float32)
                + b2_ref[...])

    F = agg_a.shape[1]
    IN = x.shape[1]
    return pl.pallas_call(
        body,
        grid=(nblk,),
        in_specs=[
            pl.BlockSpec((R, IN), lambda i: (i, 0)),
            pl.BlockSpec((R, F), lambda i: (i, 0)),
            pl.BlockSpec((R, F), lambda i: (i, 0)),
            pl.BlockSpec((R, 1), lambda i: (i, 0)),
            pl.BlockSpec((1, 1, R), lambda i: (i, 0, 0)),
            pl.BlockSpec((IN, HID), lambda i: (0, 0)),
            pl.BlockSpec((1, HID), lambda i: (0, 0)),
            pl.BlockSpec((HID, HID), lambda i: (0, 0)),
            pl.BlockSpec((1, HID), lambda i: (0, 0)),
        ],
        out_specs=[
            pl.BlockSpec((R, HID), lambda i: (i, 0)),
            pl.BlockSpec((G, HID), lambda i: (0, 0)),
        ],
        out_shape=[jax.ShapeDtypeStruct((N, HID), jnp.float32),
                   jax.ShapeDtypeStruct((G, HID), jnp.float32)],
        scratch_shapes=[pltpu.VMEM((G, HID), jnp.float32),
                        pltpu.VMEM((G, 128), jnp.float32)],
    )(x, agg_a, agg_b, dinv_col, batch3, W1, b1, W2, b2)


def _impl(x, edge_index, batch, W1, b1, W2, b2):
    N, IN = x.shape
    E = edge_index.shape[1]
    HID = W1.shape[1]
    G = 64
    F = IN // 2
    R = 1000

    ei = edge_index.astype(jnp.int32)
    src, dst = ei[0], ei[1]

    degp = _make_deg_kernel(E, N)(dst)
    dinv_col = _dinv_call(degp, N).reshape(N, 1)
    xs_a, xs_b = _prologue_call(x, dinv_col, N, F)
    zeros = jnp.zeros((N, F), jnp.float32)
    agg_a, agg_b = _make_agg_kernel(E, N, F)(xs_a, xs_b, src, dst, zeros)

    batch3 = batch.astype(jnp.int32).reshape(N // R, 1, R)
    node, graph = _epilogue_call(
        x, agg_a, agg_b, dinv_col, batch3,
        W1, b1.reshape(1, HID), W2, b2.reshape(1, HID), N, R, G, HID)
    return (graph, node)


kernel = jax.jit(_impl)
